# Initial kernel scaffold; baseline (speedup 1.0000x reference)
import jax
import jax.numpy as jnp
from jax.experimental import pallas as pl


def kernel(relation_embeddings, weights1, weights2, bias1, bias2, unique_pairs, pair_idx, rel_ids):
    raise NotImplementedError("write your pallas kernel here")


# speedup vs baseline: 63.4293x; 63.4293x over previous
import functools

import jax
import jax.numpy as jnp
from jax import lax
from jax.experimental import pallas as pl
from jax.experimental.pallas import tpu as pltpu
from jax.experimental.pallas import tpu_sc as plsc

N_NODES = 10000
NUM_RELS = 64
NUM_CLASSES = 16
EMB_DIM = 16
RP = 16
N_TRIPLES = 160000

NC, NS, L = 2, 16, 16
NW = NC * NS
F32 = jnp.float32
I32 = jnp.int32

NTP_A = 163840
TRI_PER_W = NTP_A // NW
SB_A = 1024
HALF_L = NTP_A // 2

NTP = 163840
CH = NTP // NW
SB = 256
NSUB = CH // SB
DEN_V_OFF = 160000
DEN_SP = 320128
DEN_TRASH = 320000
BW_D = 512
BW_F = 352
N_HP = 10240
H2_HALF = 80000
H2_SP = 80128
EPS = 1e-30


def _zero_vmem_2d(ref, nrows):
    def body(i, _):
        ref[i, :] = jnp.zeros((L,), F32)
        return 0
    lax.fori_loop(0, nrows, body, 0)


def _zero_spmem_rows(sp, zbuf, zrows, start, total):
    done = 0
    while done < total:
        n = min(zrows, total - done)
        pltpu.sync_copy(zbuf.at[pl.ds(0, n)], sp.at[pl.ds(start + done, n)])
        done += n


def _zero_vmem_1d(ref, n):
    def body(i, _):
        ref[pl.ds(i * L, L)] = jnp.zeros((L,), F32)
        return 0
    lax.fori_loop(0, n // L, body, 0)


def _wid(cid, sid):
    return sid * NC + cid


def _sc_latents_kernel(rel_emb, rel_ids, pair_idx, lat_p,
                       ids_v, idx2_v, rows_v, zbuf, sp_lat, sem):
    cid = lax.axis_index("c")
    sid = lax.axis_index("s")
    t0 = _wid(cid, sid) * TRI_PER_W

    _zero_vmem_2d(zbuf, 640)

    def run_half(p):
        _zero_spmem_rows(sp_lat, zbuf, 640, sid * 5120, 5120)

        @pl.when(sid == 0)
        def _():
            pltpu.sync_copy(zbuf.at[pl.ds(0, 8)], sp_lat.at[pl.ds(HALF_L, 8)])

        plsc.subcore_barrier()

        for s in range(TRI_PER_W // SB_A):
            base = t0 + s * SB_A
            pltpu.sync_copy(rel_ids.at[pl.ds(base, SB_A)], ids_v)
            pltpu.sync_copy(pair_idx.at[pl.ds(base, SB_A)], idx2_v)

            def adj(j, _):
                v = idx2_v[pl.ds(j * L, L)] - p * HALF_L
                inh = (v >= 0) & (v < HALF_L)
                idx2_v[pl.ds(j * L, L)] = jnp.where(inh, v, HALF_L)
                return 0
            lax.fori_loop(0, SB_A // L, adj, 0, unroll=4)

            pltpu.async_copy(rel_emb.at[ids_v], rows_v, sem).wait()
            pltpu.sync_copy(rows_v, sp_lat.at[idx2_v], add=True)

        plsc.subcore_barrier()
        pltpu.sync_copy(sp_lat.at[pl.ds(sid * 5120, 5120)],
                        lat_p.at[cid, pl.ds(p * HALF_L + sid * 5120, 5120)])
        plsc.subcore_barrier()

    run_half(0)
    run_half(1)


def _latents_partials(rel_emb, rel_ids_pad, pair_idx_pad):
    mesh = plsc.VectorSubcoreMesh(core_axis_name="c", subcore_axis_name="s")
    f = pl.kernel(
        _sc_latents_kernel,
        out_type=jax.ShapeDtypeStruct((NC, NTP_A, RP), F32),
        mesh=mesh,
        scratch_types=[
            pltpu.VMEM((SB_A,), I32),
            pltpu.VMEM((SB_A,), I32),
            pltpu.VMEM((SB_A, RP), F32),
            pltpu.VMEM((640, RP), F32),
            pltpu.VMEM_SHARED((HALF_L + 8, RP), F32),
            pltpu.SemaphoreType.DMA,
        ],
        compiler_params=pltpu.CompilerParams(use_tc_tiling_on_sc=False),
    )
    return f(rel_emb, rel_ids_pad, pair_idx_pad)


BT = 2048


def _tc_softmax_body(nt, lat_ref, latT_ref, latn_ref):
    i = pl.program_id(0)
    x = lat_ref[0] + lat_ref[1]
    m = jnp.max(x, axis=1, keepdims=True)
    e = jnp.exp(x - m)
    sm = e / jnp.sum(e, axis=1, keepdims=True)
    rows = i * BT + lax.broadcasted_iota(I32, (BT, 1), 0)
    sm = jnp.where(rows < nt, sm, 0.0)
    latT_ref[...] = sm.T
    latn_ref[...] = sm


def _softmax_stage(lat_p, nt):
    return pl.pallas_call(
        functools.partial(_tc_softmax_body, nt),
        grid=(NTP_A // BT,),
        in_specs=[pl.BlockSpec((NC, BT, RP), lambda i: (0, i, 0))],
        out_specs=[pl.BlockSpec((RP, BT), lambda i: (0, i)),
                   pl.BlockSpec((BT, RP), lambda i: (i, 0))],
        out_shape=[jax.ShapeDtypeStruct((RP, NTP_A), F32),
                   jax.ShapeDtypeStruct((NTP_A, RP), F32)],
    )(lat_p)


def _sc_denom_kernel(latT, s_pad, o_pad, den_p,
                     sv, ov, lf0, lf1, kb0, kb1, totv, k0v, k0i,
                     zbuf1, sp_den, sem):
    cid = lax.axis_index("c")
    sid = lax.axis_index("s")
    t0 = _wid(cid, sid) * CH

    _zero_vmem_1d(zbuf1, 10240)
    _zero_spmem_rows(sp_den, zbuf1, 10240, sid * (DEN_SP // NS), DEN_SP // NS)
    plsc.subcore_barrier()

    totv[...] = jnp.zeros((L,), F32)

    SBC = 1024
    for sub in range(CH // SBC):
        base = t0 + sub * SBC
        pltpu.sync_copy(s_pad.at[pl.ds(base, SBC)], sv)
        pltpu.sync_copy(o_pad.at[pl.ds(base, SBC)], ov)

        pltpu.sync_copy(latT.at[0, pl.ds(base, SBC)], lf0)

        def red(j, _):
            totv[...] = totv[...] + lf0[pl.ds(j * L, L)]
            return 0
        lax.fori_loop(0, SBC // L, red, 0, unroll=4)

        for k in range(1, RP):
            lf, kb = (lf0, kb0) if k % 2 else (lf1, kb1)
            pltpu.sync_copy(latT.at[k, pl.ds(base, SBC)], lf)

            def mk_h(j, _):
                kb[pl.ds(j * L, L)] = ov[pl.ds(j * L, L)] * k
                return 0
            lax.fori_loop(0, SBC // L, mk_h, 0, unroll=4)
            pltpu.sync_copy(lf, sp_den.at[kb], add=True)

            def mk_v(j, _):
                kb[pl.ds(j * L, L)] = sv[pl.ds(j * L, L)] * k + DEN_V_OFF
                return 0
            lax.fori_loop(0, SBC // L, mk_v, 0, unroll=4)
            pltpu.sync_copy(lf, sp_den.at[kb], add=True)

    tot = jnp.sum(totv[...], axis=0)
    iota = lax.iota(I32, L)
    k0v[...] = jnp.where(iota == 0, tot, 0.0)
    k0i[...] = jnp.where(iota == 0, 0, DEN_TRASH + iota)
    pltpu.sync_copy(k0v, sp_den.at[k0i], add=True)
    k0i[...] = jnp.where(iota == 0, DEN_V_OFF, DEN_TRASH + iota)
    pltpu.sync_copy(k0v, sp_den.at[k0i], add=True)

    plsc.subcore_barrier()
    n = DEN_SP // NS
    pltpu.sync_copy(sp_den.at[pl.ds(sid * n, n)],
                    den_p.at[cid, pl.ds(sid * n, n)])
    plsc.subcore_barrier()


def _denom_stage(latT, s_pad, o_pad):
    mesh = plsc.VectorSubcoreMesh(core_axis_name="c", subcore_axis_name="s")
    f = pl.kernel(
        _sc_denom_kernel,
        out_type=jax.ShapeDtypeStruct((NC, DEN_SP), F32),
        mesh=mesh,
        scratch_types=[
            pltpu.VMEM((1024,), I32),
            pltpu.VMEM((1024,), I32),
            pltpu.VMEM((1024,), F32),
            pltpu.VMEM((1024,), F32),
            pltpu.VMEM((1024,), I32),
            pltpu.VMEM((1024,), I32),
            pltpu.VMEM((L,), F32),
            pltpu.VMEM((L,), F32),
            pltpu.VMEM((L,), I32),
            pltpu.VMEM((10240,), F32),
            pltpu.VMEM_SHARED((DEN_SP,), F32),
            pltpu.SemaphoreType.DMA,
        ],
        compiler_params=pltpu.CompilerParams(use_tc_tiling_on_sc=False,
                                             needs_layout_passes=False),
    )
    return f(latT, s_pad, o_pad)


def _sc_h_kernel(latT, s_pad, o_pad, dp0, dp1, w1f, h_p,
                 sv, ov, slocal, gcol, w1rows, d0, d1, lfv, vals,
                 band, idxb, zbuf, sp_h, sem):
    cid = lax.axis_index("c")
    sid = lax.axis_index("s")
    t0 = _wid(cid, sid) * CH

    _zero_vmem_2d(zbuf, 640)
    _zero_spmem_rows(sp_h, zbuf, 640, sid * (N_HP // NS), N_HP // NS)
    _zero_vmem_2d(band, BW_D)
    plsc.subcore_barrier()

    pltpu.sync_copy(s_pad.at[pl.ds(t0, SB)], sv)
    s_base = sv[pl.ds(0, L)][0]

    def sub_body(sub, _):
        base = t0 + sub * SB
        pltpu.sync_copy(s_pad.at[pl.ds(base, SB)], sv)
        pltpu.sync_copy(o_pad.at[pl.ds(base, SB)], ov)

        def mk_sl(j, _):
            slocal[pl.ds(j * L, L)] = sv[pl.ds(j * L, L)] - s_base
            return 0
        lax.fori_loop(0, SB // L, mk_sl, 0)

        for k in range(RP):
            def mk_g(j, _):
                gcol[pl.ds(k * SB + j * L, L)] = ov[pl.ds(j * L, L)] * k
                return 0
            lax.fori_loop(0, SB // L, mk_g, 0)

        cps = []
        for k in range(RP):
            gslice = gcol.at[pl.ds(k * SB, SB)]
            cps.append(pltpu.async_copy(
                w1f.at[gslice], w1rows.at[pl.ds(k * SB, SB)], sem))
            cps.append(pltpu.async_copy(
                dp0.at[gslice], d0.at[pl.ds(k * SB, SB)], sem))
            cps.append(pltpu.async_copy(
                dp1.at[gslice], d1.at[pl.ds(k * SB, SB)], sem))
            cps.append(pltpu.async_copy(
                latT.at[k, pl.ds(base, SB)], lfv.at[pl.ds(k * SB, SB)], sem))
        for cp in cps:
            cp.wait()

        def mk_val(j, _):
            sl = pl.ds(j * L, L)
            vals[sl] = lfv[sl] / (d0[sl] + d1[sl] + EPS)
            return 0
        lax.fori_loop(0, (RP * SB) // L, mk_val, 0, unroll=4)

        def blk(j, _):
            slv = slocal[pl.ds(j * L, L)]
            valvs = [vals[pl.ds(k * SB + j * L, L)] for k in range(RP)]
            for i in range(L):
                t = j * L + i
                slt = slv[i]
                acc = band[slt, :]
                terms = [w1rows[k * SB + t, :] * valvs[k][i]
                         for k in range(RP)]
                while len(terms) > 1:
                    terms = [terms[q] + terms[q + 1]
                             for q in range(0, len(terms) - 1, 2)] + \
                            ([terms[-1]] if len(terms) % 2 else [])
                band[slt, :] = acc + terms[0]
            return 0
        lax.fori_loop(0, SB // L, blk, 0)
        return 0

    lax.fori_loop(0, NSUB, sub_body, 0)

    iota = lax.iota(I32, L)

    def mk_idx(j, _):
        v = s_base + j * L + iota
        idxb[pl.ds(j * L, L)] = jnp.minimum(v, N_NODES)
        return 0
    lax.fori_loop(0, BW_D // L, mk_idx, 0)
    pltpu.sync_copy(band, sp_h.at[idxb], add=True)

    plsc.subcore_barrier()
    n = N_HP // NS
    pltpu.sync_copy(sp_h.at[pl.ds(sid * n, n)],
                    h_p.at[cid, pl.ds(sid * n, n)])
    plsc.subcore_barrier()


def _h_stage(latT, s_pad, o_pad, dp0, dp1, w1f):
    mesh = plsc.VectorSubcoreMesh(core_axis_name="c", subcore_axis_name="s")
    f = pl.kernel(
        _sc_h_kernel,
        out_type=jax.ShapeDtypeStruct((NC, N_HP, EMB_DIM), F32),
        mesh=mesh,
        scratch_types=[
            pltpu.VMEM((SB,), I32),
            pltpu.VMEM((SB,), I32),
            pltpu.VMEM((SB,), I32),
            pltpu.VMEM((RP * SB,), I32),
            pltpu.VMEM((RP * SB, EMB_DIM), F32),
            pltpu.VMEM((RP * SB,), F32),
            pltpu.VMEM((RP * SB,), F32),
            pltpu.VMEM((RP * SB,), F32),
            pltpu.VMEM((RP * SB,), F32),
            pltpu.VMEM((BW_D, EMB_DIM), F32),
            pltpu.VMEM((BW_D,), I32),
            pltpu.VMEM((640, RP), F32),
            pltpu.VMEM_SHARED((N_HP, EMB_DIM), F32),
            pltpu.SemaphoreType.DMA,
        ],
        compiler_params=pltpu.CompilerParams(use_tc_tiling_on_sc=False),
    )
    return f(latT, s_pad, o_pad, dp0, dp1, w1f)


def _tc_relu_body(hp_ref, b_ref, out_ref):
    x = hp_ref[0, :N_NODES] + hp_ref[1, :N_NODES] + b_ref[...]
    out_ref[...] = jnp.maximum(x, 0.0)


def _relu_stage(h_p, bias1_r):
    return pl.pallas_call(
        _tc_relu_body,
        in_specs=[pl.BlockSpec((NC, N_HP, EMB_DIM), lambda: (0, 0, 0)),
                  pl.BlockSpec((1, EMB_DIM), lambda: (0, 0))],
        out_specs=pl.BlockSpec((N_NODES, EMB_DIM), lambda: (0, 0)),
        out_shape=jax.ShapeDtypeStruct((N_NODES, EMB_DIM), F32),
    )(h_p, bias1_r)


def _sc_h2_kernel(lat_n, s_pad, o_pad, dp0, dp1, h_cmb, h2b,
                  sv, ov, slocal, latrows, hrows, bands, dd0, dden,
                  kib, sem):
    cid = lax.axis_index("c")
    sid = lax.axis_index("s")
    wid = _wid(cid, sid)
    t0 = wid * CH
    iota = lax.iota(I32, L)

    _zero_vmem_2d(bands, RP * BW_F)

    pltpu.sync_copy(s_pad.at[pl.ds(t0, SB)], sv)
    s_base = sv[pl.ds(0, L)][0]

    def mk_ki(i, _):
        kib[pl.ds(i * L, L)] = (s_base + i) * iota + DEN_V_OFF
        return 0
    lax.fori_loop(0, BW_F, mk_ki, 0, unroll=4)
    pltpu.async_copy(dp0.at[kib], dd0, sem).wait()

    def cp_d(i, _):
        dden[i, :] = dd0[pl.ds(i * L, L)]
        return 0
    lax.fori_loop(0, BW_F, cp_d, 0, unroll=4)
    pltpu.async_copy(dp1.at[kib], dd0, sem).wait()

    def add_d(i, _):
        dden[i, :] = dden[i, :] + dd0[pl.ds(i * L, L)] + EPS
        return 0
    lax.fori_loop(0, BW_F, add_d, 0, unroll=4)

    def sub_body(sub, _):
        base = t0 + sub * SB
        pltpu.sync_copy(s_pad.at[pl.ds(base, SB)], sv)
        pltpu.sync_copy(o_pad.at[pl.ds(base, SB)], ov)

        def mk_sl(j, _):
            slocal[pl.ds(j * L, L)] = sv[pl.ds(j * L, L)] - s_base
            return 0
        lax.fori_loop(0, SB // L, mk_sl, 0)

        cp1 = pltpu.async_copy(h_cmb.at[ov], hrows, sem)
        cp2 = pltpu.async_copy(lat_n.at[pl.ds(base, SB)], latrows, sem)
        cp1.wait()
        cp2.wait()

        def blk(j, _):
            slv = slocal[pl.ds(j * L, L)]
            for i in range(L):
                t = j * L + i
                slt = slv[i]
                latv = latrows[t, :]
                denv = dden[slt, :]
                val = latv / denv
                hrow = hrows[t, :]
                for k in range(RP):
                    r = k * BW_F + slt
                    bands[r, :] = bands[r, :] + hrow * val[k]
            return 0
        lax.fori_loop(0, SB // L, blk, 0)
        return 0

    lax.fori_loop(0, NSUB, sub_body, 0)

    pltpu.sync_copy(bands, h2b.at[wid])


def _h2_stage(lat_n, s_pad, o_pad, dp0, dp1, h_cmb):
    mesh = plsc.VectorSubcoreMesh(core_axis_name="c", subcore_axis_name="s")
    f = pl.kernel(
        _sc_h2_kernel,
        out_type=jax.ShapeDtypeStruct((NW, RP * BW_F, EMB_DIM), F32),
        mesh=mesh,
        scratch_types=[
            pltpu.VMEM((SB,), I32),
            pltpu.VMEM((SB,), I32),
            pltpu.VMEM((SB,), I32),
            pltpu.VMEM((SB, EMB_DIM), F32),
            pltpu.VMEM((SB, EMB_DIM), F32),
            pltpu.VMEM((RP * BW_F, EMB_DIM), F32),
            pltpu.VMEM((L * BW_F,), F32),
            pltpu.VMEM((BW_F, L), F32),
            pltpu.VMEM((L * BW_F,), I32),
            pltpu.SemaphoreType.DMA,
        ],
        compiler_params=pltpu.CompilerParams(use_tc_tiling_on_sc=False,
                                             needs_layout_passes=False),
    )
    return f(lat_n, s_pad, o_pad, dp0, dp1, h_cmb)


def _sc_h2scat_kernel(h2b, s_pad, h2_p, bandbuf, sbv, idxo, zbuf,
                      sp_h2, sem):
    cid = lax.axis_index("c")
    sid = lax.axis_index("s")
    wid = _wid(cid, sid)
    iota = lax.iota(I32, L)

    _zero_vmem_2d(zbuf, 640)
    pltpu.sync_copy(s_pad.at[pl.ds(wid * CH, L)], sbv)
    s_base = sbv[...][0]

    for p in range(2):
        _zero_spmem_rows(sp_h2, zbuf, 640, sid * (H2_SP // NS), H2_SP // NS)
        plsc.subcore_barrier()
        for k in range(RP):
            pltpu.sync_copy(h2b.at[wid, pl.ds(k * BW_F, BW_F)], bandbuf)

            def mk_io(j, _):
                v = (s_base + j * L + iota) * k - p * H2_HALF
                ok = (v >= 0) & (v < H2_HALF)
                idxo[pl.ds(j * L, L)] = jnp.where(ok, v, H2_HALF)
                return 0
            lax.fori_loop(0, BW_F // L, mk_io, 0)
            pltpu.sync_copy(bandbuf, sp_h2.at[idxo], add=True)
        plsc.subcore_barrier()
        n = H2_HALF // NS
        pltpu.sync_copy(sp_h2.at[pl.ds(sid * n, n)],
                        h2_p.at[cid, pl.ds(p * H2_HALF + sid * n, n)])
        plsc.subcore_barrier()


def _h2scat_stage(h2b, s_pad):
    mesh = plsc.VectorSubcoreMesh(core_axis_name="c", subcore_axis_name="s")
    f = pl.kernel(
        _sc_h2scat_kernel,
        out_type=jax.ShapeDtypeStruct((NC, 2 * H2_HALF, EMB_DIM), F32),
        mesh=mesh,
        scratch_types=[
            pltpu.VMEM((BW_F, EMB_DIM), F32),
            pltpu.VMEM((L,), I32),
            pltpu.VMEM((BW_F,), I32),
            pltpu.VMEM((640, RP), F32),
            pltpu.VMEM_SHARED((H2_SP, EMB_DIM), F32),
            pltpu.SemaphoreType.DMA,
        ],
        compiler_params=pltpu.CompilerParams(use_tc_tiling_on_sc=False),
    )
    return f(h2b, s_pad)


BN = 1000


def _tc_out_body(h2_ref, w2_ref, b_ref, out_ref):
    acc = jnp.zeros((BN, NUM_CLASSES), F32)
    for r in range(RP):
        x = h2_ref[0, r] + h2_ref[1, r]
        acc = acc + jnp.dot(x, w2_ref[r], preferred_element_type=F32)
    out_ref[...] = acc + b_ref[...]


def _out_stage(h2_r, w2, bias2_r):
    return pl.pallas_call(
        _tc_out_body,
        grid=(N_NODES // BN,),
        in_specs=[
            pl.BlockSpec((NC, RP, BN, EMB_DIM), lambda i: (0, 0, i, 0)),
            pl.BlockSpec((RP, EMB_DIM, NUM_CLASSES), lambda i: (0, 0, 0)),
            pl.BlockSpec((1, NUM_CLASSES), lambda i: (0, 0)),
        ],
        out_specs=pl.BlockSpec((BN, NUM_CLASSES), lambda i: (i, 0)),
        out_shape=jax.ShapeDtypeStruct((N_NODES, NUM_CLASSES), F32),
    )(h2_r, w2, bias2_r)


def kernel(relation_embeddings, weights1, weights2, bias1, bias2,
           unique_pairs, pair_idx, rel_ids):
    nt = unique_pairs.shape[0]

    npad = NTP_A - N_TRIPLES
    pair_idx_pad = jnp.concatenate(
        [pair_idx, jnp.full((npad,), nt, I32)])
    rel_ids_pad = jnp.concatenate([rel_ids, jnp.zeros((npad,), I32)])

    ppad = NTP - nt
    s_pad = jnp.concatenate(
        [unique_pairs[:, 0], jnp.full((ppad,), N_NODES - 1, I32)])
    o_pad = jnp.concatenate([unique_pairs[:, 1], jnp.zeros((ppad,), I32)])
    w1f = weights1.reshape(RP * N_NODES, EMB_DIM)

    lat_p = _latents_partials(relation_embeddings, rel_ids_pad, pair_idx_pad)
    latT, lat_n = _softmax_stage(lat_p, nt)
    den_p = _denom_stage(latT, s_pad, o_pad)
    dp0, dp1 = den_p[0], den_p[1]
    h_p = _h_stage(latT, s_pad, o_pad, dp0, dp1, w1f)
    h_cmb = _relu_stage(h_p, bias1.reshape(1, EMB_DIM))
    h2b = _h2_stage(lat_n, s_pad, o_pad, dp0, dp1, h_cmb)
    h2_p = _h2scat_stage(h2b, s_pad)
    h2_r = h2_p.reshape(NC, RP, N_NODES, EMB_DIM)
    out = _out_stage(h2_r, weights2, bias2.reshape(1, NUM_CLASSES))
    return out
